# baseline (speedup 1.0000x reference)
import numpy as np
import jax
import jax.numpy as jnp
from jax.experimental import pallas as pl

_OUT_CH = 192
_FEAT_DIM = _OUT_CH // 6
_PB = 64

_TWO_OVER_PI = 0.6366197723675814
_PIO2_HI = np.float32(1.57079637050628662109375)
_PIO2_LO = np.float32(-4.37113900018624283e-8)
_MAGIC = np.float32(1.5 * 2.0**23)

_S1 = np.float32(-1.6666654611e-1)
_S2 = np.float32(8.3321608736e-3)
_S3 = np.float32(-1.9515295891e-4)
_C0 = np.float32(2.443315711809948e-5)
_C1 = np.float32(-1.388731625493765e-3)
_C2 = np.float32(4.166664568298827e-2)


def _sincos(t):
    kb = t * np.float32(_TWO_OVER_PI) + _MAGIC
    bits = jax.lax.bitcast_convert_type(kb, jnp.int32) - np.int32(0x4B400000)
    k = bits.astype(jnp.float32)
    y = t - k * _PIO2_HI
    y = y - k * _PIO2_LO
    z = y * y
    ps = _S3 * z + _S2
    ps = ps * z + _S1
    s = y + (y * z) * ps
    pc = _C0 * z + _C1
    pc = pc * z + _C2
    c = (z * z) * pc + (np.float32(1.0) - np.float32(0.5) * z)
    swap = (bits & 1) == 1
    sin_base = jnp.where(swap, c, s)
    cos_base = jnp.where(swap, s, c)
    sin_flip = (bits & 2) << 30
    cos_flip = ((bits + 1) & 2) << 30
    sin_t = jax.lax.bitcast_convert_type(
        jax.lax.bitcast_convert_type(sin_base, jnp.int32) ^ sin_flip, jnp.float32)
    cos_t = jax.lax.bitcast_convert_type(
        jax.lax.bitcast_convert_type(cos_base, jnp.int32) ^ cos_flip, jnp.float32)
    return sin_t, cos_t


def _pospool_kernel(s_ref, r_ref, x_ref, o_ref):
    s = s_ref[...].reshape(1, _FEAT_DIM, 1)
    fd = _FEAT_DIM
    pb, ns = r_ref.shape[2], r_ref.shape[3]
    r = r_ref[...].reshape(1, 3, pb * ns)
    x = x_ref[...].reshape(1, x_ref.shape[1], pb * ns)
    for i in range(3):
        t = r[:, i : i + 1, :] * s
        sin_t, cos_t = _sincos(t)
        xs = x[:, 2 * i * fd : (2 * i + 1) * fd, :]
        o_ref[:, 2 * i * fd : (2 * i + 1) * fd, :, :] = (
            xs * sin_t + sin_t).reshape(1, fd, pb, ns)
        xc = x[:, (2 * i + 1) * fd : (2 * i + 2) * fd, :]
        o_ref[:, (2 * i + 1) * fd : (2 * i + 2) * fd, :, :] = (
            xc * cos_t + cos_t).reshape(1, fd, pb, ns)


def kernel(re_xyz, x):
    B, _, npoint, nsample = re_xyz.shape
    C = x.shape[1]

    fr = jnp.arange(_FEAT_DIM, dtype=jnp.float32)
    dim_mat = jnp.power(jnp.float32(500.0), (1.0 / _FEAT_DIM) * fr)
    scale = (50.0 / dim_mat).reshape(1, _FEAT_DIM, 1, 1)

    nblk = npoint // _PB
    return pl.pallas_call(
        _pospool_kernel,
        grid=(B, nblk),
        in_specs=[
            pl.BlockSpec((1, _FEAT_DIM, 1, 1), lambda b, n: (0, 0, 0, 0)),
            pl.BlockSpec((1, 3, _PB, nsample), lambda b, n: (b, 0, n, 0)),
            pl.BlockSpec((1, C, _PB, nsample), lambda b, n: (b, 0, n, 0)),
        ],
        out_specs=pl.BlockSpec((1, C, _PB, nsample), lambda b, n: (b, 0, n, 0)),
        out_shape=jax.ShapeDtypeStruct((B, C, npoint, nsample), jnp.float32),
    )(scale, re_xyz, x)


# speedup vs baseline: 1.3991x; 1.3991x over previous
import numpy as np
import jax
import jax.numpy as jnp
from jax.experimental import pallas as pl

_OUT_CH = 192
_FEAT_DIM = _OUT_CH // 6
_QB = 32

_TWO_OVER_PI = 0.6366197723675814
_PIO2_HI = np.float32(1.57079637050628662109375)
_PIO2_LO = np.float32(-4.37113900018624283e-8)
_MAGIC = np.float32(1.5 * 2.0**23)

_S1 = np.float32(-1.6666654611e-1)
_S2 = np.float32(8.3321608736e-3)
_S3 = np.float32(-1.9515295891e-4)
_C0 = np.float32(2.443315711809948e-5)
_C1 = np.float32(-1.388731625493765e-3)
_C2 = np.float32(4.166664568298827e-2)


def _sincos(t):
    kb = t * np.float32(_TWO_OVER_PI) + _MAGIC
    bits = jax.lax.bitcast_convert_type(kb, jnp.int32) - np.int32(0x4B400000)
    k = bits.astype(jnp.float32)
    y = t - k * _PIO2_HI
    y = y - k * _PIO2_LO
    z = y * y
    ps = _S3 * z + _S2
    ps = ps * z + _S1
    s = y + (y * z) * ps
    pc = _C0 * z + _C1
    pc = pc * z + _C2
    c = (z * z) * pc + (np.float32(1.0) - np.float32(0.5) * z)
    swap = (bits & 1) == 1
    sin_base = jnp.where(swap, c, s)
    cos_base = jnp.where(swap, s, c)
    sin_flip = (bits & 2) << 30
    cos_flip = ((bits + 1) & 2) << 30
    sin_t = jax.lax.bitcast_convert_type(
        jax.lax.bitcast_convert_type(sin_base, jnp.int32) ^ sin_flip, jnp.float32)
    cos_t = jax.lax.bitcast_convert_type(
        jax.lax.bitcast_convert_type(cos_base, jnp.int32) ^ cos_flip, jnp.float32)
    return sin_t, cos_t


def _pospool_kernel(s_ref, r_ref, x_ref, o_ref):
    s = s_ref[...]
    fd = _FEAT_DIM
    for i in range(3):
        t = r_ref[:, i : i + 1, :, :] * s
        sin_t, cos_t = _sincos(t)
        xs = x_ref[:, 2 * i * fd : (2 * i + 1) * fd, :, :]
        o_ref[:, 2 * i * fd : (2 * i + 1) * fd, :, :] = xs * sin_t + sin_t
        xc = x_ref[:, (2 * i + 1) * fd : (2 * i + 2) * fd, :, :]
        o_ref[:, (2 * i + 1) * fd : (2 * i + 2) * fd, :, :] = xc * cos_t + cos_t


def kernel(re_xyz, x):
    B, _, npoint, nsample = re_xyz.shape
    C = x.shape[1]
    n_rows = (npoint * nsample) // 128
    r = re_xyz.reshape(B, 3, n_rows, 128)
    xf = x.reshape(B, C, n_rows, 128)

    fr = jnp.arange(_FEAT_DIM, dtype=jnp.float32)
    dim_mat = jnp.power(jnp.float32(500.0), (1.0 / _FEAT_DIM) * fr)
    scale = (50.0 / dim_mat).reshape(1, _FEAT_DIM, 1, 1)

    nblk = n_rows // _QB
    out = pl.pallas_call(
        _pospool_kernel,
        grid=(B, nblk),
        in_specs=[
            pl.BlockSpec((1, _FEAT_DIM, 1, 1), lambda b, n: (0, 0, 0, 0)),
            pl.BlockSpec((1, 3, _QB, 128), lambda b, n: (b, 0, n, 0)),
            pl.BlockSpec((1, C, _QB, 128), lambda b, n: (b, 0, n, 0)),
        ],
        out_specs=pl.BlockSpec((1, C, _QB, 128), lambda b, n: (b, 0, n, 0)),
        out_shape=jax.ShapeDtypeStruct((B, C, n_rows, 128), jnp.float32),
    )(scale, r, xf)
    return out.reshape(B, C, npoint, nsample)
